# Initial kernel scaffold; baseline (speedup 1.0000x reference)
import jax
import jax.numpy as jnp
from jax.experimental import pallas as pl


def kernel(x, edge_index, params):
    raise NotImplementedError("write your pallas kernel here")


# speedup vs baseline: 1.3299x; 1.3299x over previous
import jax
import jax.numpy as jnp
from jax import lax
from jax.experimental import pallas as pl
from jax.experimental.pallas import tpu as pltpu

N = 10000
DIM = 128
E = 320000
EB = 1024
EPAD = 327680
DEGW = 8
BLK = 1000


def _seg_body(sidx, didx, ta, tb, outa, outb, deg):
    step = pl.program_id(0)

    @pl.when(step == 0)
    def _():
        outa[...] = jnp.zeros_like(outa)
        outb[...] = jnp.zeros_like(outb)
        deg[...] = jnp.zeros_like(deg)

    def edge(i, c):
        s = sidx[i]
        d = didx[i]
        ra = ta[pl.ds(s, 1), :]
        rb = tb[pl.ds(s, 1), :]
        one = jnp.full((1, DEGW), 1.0, jnp.float32) * jnp.where(
            s < N, 1.0, 0.0).astype(jnp.float32)
        outa[pl.ds(d, 1), :] += ra
        outb[pl.ds(d, 1), :] += rb
        deg[pl.ds(d, 1), :] += one
        return c

    lax.fori_loop(0, EB, edge, 0)


def _run_seg(ta, tb, src, dst, interpret=False):
    ta = jnp.concatenate([ta, jnp.zeros((1, DIM), jnp.float32)])
    tb = jnp.concatenate([tb, jnp.zeros((1, DIM), jnp.float32)])
    src = jnp.concatenate([src, jnp.full((EPAD - E,), N, jnp.int32)])
    dst = jnp.concatenate([dst, jnp.zeros((EPAD - E,), jnp.int32)])
    f = pl.pallas_call(
        _seg_body,
        grid=(EPAD // EB,),
        in_specs=[
            pl.BlockSpec((EB,), lambda i: (i,), memory_space=pltpu.SMEM),
            pl.BlockSpec((EB,), lambda i: (i,), memory_space=pltpu.SMEM),
            pl.BlockSpec((N + 1, DIM), lambda i: (0, 0)),
            pl.BlockSpec((N + 1, DIM), lambda i: (0, 0)),
        ],
        out_specs=[
            pl.BlockSpec((N, DIM), lambda i: (0, 0)),
            pl.BlockSpec((N, DIM), lambda i: (0, 0)),
            pl.BlockSpec((N, DEGW), lambda i: (0, 0)),
        ],
        out_shape=[
            jax.ShapeDtypeStruct((N, DIM), jnp.float32),
            jax.ShapeDtypeStruct((N, DIM), jnp.float32),
            jax.ShapeDtypeStruct((N, DEGW), jnp.float32),
        ],
        interpret=interpret,
    )
    return f(src, dst, ta, tb)


def _lk(v):
    return jnp.where(v >= 0.0, v, 0.01 * v)


def _dot(a, b):
    return jnp.dot(a, b, preferred_element_type=jnp.float32)


def _row(w):
    return pl.BlockSpec((BLK, w), lambda i: (i, 0))


def _full(shape):
    return pl.BlockSpec(shape, lambda i: (0,) * len(shape))


def _tc_pre_body(x, aW1, ab1, aW2, ab2, cW, mask_o, xp_o, gp_o):
    h = _lk(_dot(x[:], aW1[:]) + ab1[:])
    m = jax.nn.sigmoid(_dot(h, aW2[:]) + ab2[:])
    xp = x[:] * m
    mask_o[:] = m
    xp_o[:] = xp
    gp_o[:] = _dot(xp, cW[:])


def _run_pre(x, p):
    f = pl.pallas_call(
        _tc_pre_body,
        grid=(N // BLK,),
        in_specs=[
            _row(DIM),
            _full((DIM, 64)), _full((1, 64)), _full((64, DIM)), _full((1, DIM)),
            _full((DIM, DIM)),
        ],
        out_specs=[_row(DIM)] * 3,
        out_shape=[jax.ShapeDtypeStruct((N, DIM), jnp.float32)] * 3,
    )
    return f(x, p['apg_W1'], p['apg_b1'].reshape(1, 64),
             p['apg_W2'], p['apg_b2'].reshape(1, DIM), p['conv1_W'])


def _tc_scale_body(gp, d, g_o):
    dis = lax.rsqrt(d[:, 0:1] + 1.0)
    g_o[:] = gp[:] * dis


def _run_scale(gp, d):
    f = pl.pallas_call(
        _tc_scale_body,
        grid=(N // BLK,),
        in_specs=[_row(DIM), _row(DEGW)],
        out_specs=[_row(DIM)],
        out_shape=[jax.ShapeDtypeStruct((N, DIM), jnp.float32)],
    )
    return f(gp, d)[0]


def _tc_mid_body(sa, sb, g1, d, sW1, sb1_, sW2, sb2_, cb, c2W,
                 sp1_o, h1_o, g2p_o):
    deg = d[:, 0:1]
    dis = lax.rsqrt(deg + 1.0)
    agg = sa[:] / (deg + 1e-6)
    sp1 = _dot(_lk(_dot(agg, sW1[:]) + sb1_[:]), sW2[:]) + sb2_[:]
    h1 = _lk(dis * (sb[:] + g1[:]) + cb[:] + sp1)
    sp1_o[:] = sp1
    h1_o[:] = h1
    g2p_o[:] = _dot(h1, c2W[:]) * dis


def _run_mid(sa, sb, g1, d, p):
    f = pl.pallas_call(
        _tc_mid_body,
        grid=(N // BLK,),
        in_specs=[
            _row(DIM), _row(DIM), _row(DIM), _row(DEGW),
            _full((DIM, 64)), _full((1, 64)), _full((64, DIM)), _full((1, DIM)),
            _full((1, DIM)), _full((DIM, DIM)),
        ],
        out_specs=[_row(DIM)] * 3,
        out_shape=[jax.ShapeDtypeStruct((N, DIM), jnp.float32)] * 3,
    )
    return f(sa, sb, g1, d,
             p['spg1_W1'], p['spg1_b1'].reshape(1, 64),
             p['spg1_W2'], p['spg1_b2'].reshape(1, DIM),
             p['conv1_b'].reshape(1, DIM), p['conv2_W'])


def _tc_post_body(sa, sb, g2, d, sW1, sb1_, sW2, sb2_, cb,
                  dW1, db1, dW2, db2, clW, clb, avW1, avb1, avW2, avb2,
                  cfW1, cfb1, cfW2, cfb2,
                  sp2_o, h2_o, xr_o, lg_o, ad_o, cf_o):
    deg = d[:, 0:1]
    dis = lax.rsqrt(deg + 1.0)
    agg = sa[:] / (deg + 1e-6)
    sp2 = _dot(_lk(_dot(agg, sW1[:]) + sb1_[:]), sW2[:]) + sb2_[:]
    h2 = _lk(dis * (sb[:] + g2[:]) + cb[:] + sp2)
    sp2_o[:] = sp2
    h2_o[:] = h2
    xr_o[:] = _dot(_lk(_dot(h2, dW1[:]) + db1[:]), dW2[:]) + db2[:]
    lg_o[:] = _dot(h2, clW[:]) + clb[:]
    ad_o[:] = _dot(_lk(_dot(h2, avW1[:]) + avb1[:]), avW2[:]) + avb2[:]
    cf_o[:] = _dot(_lk(_dot(h2, cfW1[:]) + cfb1[:]), cfW2[:]) + cfb2[:]


def _run_post(sa, sb, g2, d, p):
    f = pl.pallas_call(
        _tc_post_body,
        grid=(N // BLK,),
        in_specs=[
            _row(DIM), _row(DIM), _row(DIM), _row(DEGW),
            _full((DIM, 64)), _full((1, 64)), _full((64, DIM)), _full((1, DIM)),
            _full((1, DIM)),
            _full((DIM, DIM)), _full((1, DIM)), _full((DIM, DIM)), _full((1, DIM)),
            _full((DIM, 2)), _full((1, 2)),
            _full((DIM, 64)), _full((1, 64)), _full((64, 2)), _full((1, 2)),
            _full((DIM, DIM)), _full((1, DIM)), _full((DIM, DIM)), _full((1, DIM)),
        ],
        out_specs=[_row(DIM), _row(DIM), _row(DIM), _row(2), _row(2), _row(DIM)],
        out_shape=[
            jax.ShapeDtypeStruct((N, DIM), jnp.float32),
            jax.ShapeDtypeStruct((N, DIM), jnp.float32),
            jax.ShapeDtypeStruct((N, DIM), jnp.float32),
            jax.ShapeDtypeStruct((N, 2), jnp.float32),
            jax.ShapeDtypeStruct((N, 2), jnp.float32),
            jax.ShapeDtypeStruct((N, DIM), jnp.float32),
        ],
    )
    return f(sa, sb, g2, d,
             p['spg2_W1'], p['spg2_b1'].reshape(1, 64),
             p['spg2_W2'], p['spg2_b2'].reshape(1, DIM),
             p['conv2_b'].reshape(1, DIM),
             p['dec_W1'], p['dec_b1'].reshape(1, DIM),
             p['dec_W2'], p['dec_b2'].reshape(1, DIM),
             p['cls_W'], p['cls_b'].reshape(1, 2),
             p['adv_W1'], p['adv_b1'].reshape(1, 64),
             p['adv_W2'], p['adv_b2'].reshape(1, 2),
             p['cf_W1'], p['cf_b1'].reshape(1, DIM),
             p['cf_W2'], p['cf_b2'].reshape(1, DIM))


def kernel(x, edge_index, params):
    src, dst = edge_index[0], edge_index[1]
    mask, xp, g1p = _run_pre(x, params)
    sa1, g1pre, d = _run_seg(xp, g1p, src, dst)
    g1 = _run_scale(g1p, d)
    _, sb1, _ = _run_seg(xp, g1, src, dst)
    sp1, h1, g2 = _run_mid(sa1, sb1, g1, d, params)
    sa2, sb2, _ = _run_seg(h1, g2, src, dst)
    sp2, h2, x_recon, logits, adv_logits, cf = _run_post(sa2, sb2, g2, d,
                                                         params)
    return (x_recon, h2, mask, sp1, sp2, logits, adv_logits, cf)


# speedup vs baseline: 1.4437x; 1.0855x over previous
import jax
import jax.numpy as jnp
from jax import lax
from jax.experimental import pallas as pl
from jax.experimental.pallas import tpu as pltpu

N = 10000
DIM = 128
E = 320000
EB = 1024
EPAD = 327680
DEGW = 8
BLK = 1000


def _seg_body(sidx, didx, ta, tb, outa, outb):
    step = pl.program_id(0)

    @pl.when(step == 0)
    def _():
        outa[...] = jnp.zeros_like(outa)
        outb[...] = jnp.zeros_like(outb)

    def edge(i, c):
        s = sidx[i]
        d = didx[i]
        ra = ta[pl.ds(s, 1), :]
        rb = tb[pl.ds(s, 1), :]
        outa[pl.ds(d, 1), :] += ra
        outb[pl.ds(d, 1), :] += rb
        return c

    lax.fori_loop(0, EB, edge, 0)


def _deg_body(sidx, didx, deg):
    step = pl.program_id(0)

    @pl.when(step == 0)
    def _():
        deg[...] = jnp.zeros_like(deg)

    def edge(i, c):
        s = sidx[i]
        d = didx[i]
        one = jnp.full((1, DEGW), 1.0, jnp.float32) * jnp.where(
            s < N, 1.0, 0.0).astype(jnp.float32)
        deg[pl.ds(d, 1), :] += one
        return c

    lax.fori_loop(0, EB, edge, 0)


def _pad_edges(src, dst):
    src = jnp.concatenate([src, jnp.full((EPAD - E,), N, jnp.int32)])
    dst = jnp.concatenate([dst, jnp.zeros((EPAD - E,), jnp.int32)])
    return src, dst


def _run_deg(src, dst, interpret=False):
    src, dst = _pad_edges(src, dst)
    f = pl.pallas_call(
        _deg_body,
        grid=(EPAD // EB,),
        in_specs=[
            pl.BlockSpec((EB,), lambda i: (i,), memory_space=pltpu.SMEM),
            pl.BlockSpec((EB,), lambda i: (i,), memory_space=pltpu.SMEM),
        ],
        out_specs=[pl.BlockSpec((N, DEGW), lambda i: (0, 0))],
        out_shape=[jax.ShapeDtypeStruct((N, DEGW), jnp.float32)],
        interpret=interpret,
    )
    return f(src, dst)[0]


def _run_seg(ta, tb, src, dst, interpret=False):
    ta = jnp.concatenate([ta, jnp.zeros((1, DIM), jnp.float32)])
    tb = jnp.concatenate([tb, jnp.zeros((1, DIM), jnp.float32)])
    src, dst = _pad_edges(src, dst)
    f = pl.pallas_call(
        _seg_body,
        grid=(EPAD // EB,),
        in_specs=[
            pl.BlockSpec((EB,), lambda i: (i,), memory_space=pltpu.SMEM),
            pl.BlockSpec((EB,), lambda i: (i,), memory_space=pltpu.SMEM),
            pl.BlockSpec((N + 1, DIM), lambda i: (0, 0)),
            pl.BlockSpec((N + 1, DIM), lambda i: (0, 0)),
        ],
        out_specs=[
            pl.BlockSpec((N, DIM), lambda i: (0, 0)),
            pl.BlockSpec((N, DIM), lambda i: (0, 0)),
        ],
        out_shape=[
            jax.ShapeDtypeStruct((N, DIM), jnp.float32),
            jax.ShapeDtypeStruct((N, DIM), jnp.float32),
        ],
        interpret=interpret,
    )
    return f(src, dst, ta, tb)


def _lk(v):
    return jnp.where(v >= 0.0, v, 0.01 * v)


def _dot(a, b):
    return jnp.dot(a, b, preferred_element_type=jnp.float32)


def _row(w):
    return pl.BlockSpec((BLK, w), lambda i: (i, 0))


def _full(shape):
    return pl.BlockSpec(shape, lambda i: (0,) * len(shape))


def _tc_pre_body(x, aW1, ab1, aW2, ab2, cW, mask_o, xp_o, gp_o):
    h = _lk(_dot(x[:], aW1[:]) + ab1[:])
    m = jax.nn.sigmoid(_dot(h, aW2[:]) + ab2[:])
    xp = x[:] * m
    mask_o[:] = m
    xp_o[:] = xp
    gp_o[:] = _dot(xp, cW[:])


def _run_pre(x, p):
    f = pl.pallas_call(
        _tc_pre_body,
        grid=(N // BLK,),
        in_specs=[
            _row(DIM),
            _full((DIM, 64)), _full((1, 64)), _full((64, DIM)), _full((1, DIM)),
            _full((DIM, DIM)),
        ],
        out_specs=[_row(DIM)] * 3,
        out_shape=[jax.ShapeDtypeStruct((N, DIM), jnp.float32)] * 3,
    )
    return f(x, p['apg_W1'], p['apg_b1'].reshape(1, 64),
             p['apg_W2'], p['apg_b2'].reshape(1, DIM), p['conv1_W'])


def _tc_scale_body(gp, d, g_o):
    dis = lax.rsqrt(d[:, 0:1] + 1.0)
    g_o[:] = gp[:] * dis


def _run_scale(gp, d):
    f = pl.pallas_call(
        _tc_scale_body,
        grid=(N // BLK,),
        in_specs=[_row(DIM), _row(DEGW)],
        out_specs=[_row(DIM)],
        out_shape=[jax.ShapeDtypeStruct((N, DIM), jnp.float32)],
    )
    return f(gp, d)[0]


def _tc_mid_body(sa, sb, g1, d, sW1, sb1_, sW2, sb2_, cb, c2W,
                 sp1_o, h1_o, g2p_o):
    deg = d[:, 0:1]
    dis = lax.rsqrt(deg + 1.0)
    agg = sa[:] / (deg + 1e-6)
    sp1 = _dot(_lk(_dot(agg, sW1[:]) + sb1_[:]), sW2[:]) + sb2_[:]
    h1 = _lk(dis * (sb[:] + g1[:]) + cb[:] + sp1)
    sp1_o[:] = sp1
    h1_o[:] = h1
    g2p_o[:] = _dot(h1, c2W[:]) * dis


def _run_mid(sa, sb, g1, d, p):
    f = pl.pallas_call(
        _tc_mid_body,
        grid=(N // BLK,),
        in_specs=[
            _row(DIM), _row(DIM), _row(DIM), _row(DEGW),
            _full((DIM, 64)), _full((1, 64)), _full((64, DIM)), _full((1, DIM)),
            _full((1, DIM)), _full((DIM, DIM)),
        ],
        out_specs=[_row(DIM)] * 3,
        out_shape=[jax.ShapeDtypeStruct((N, DIM), jnp.float32)] * 3,
    )
    return f(sa, sb, g1, d,
             p['spg1_W1'], p['spg1_b1'].reshape(1, 64),
             p['spg1_W2'], p['spg1_b2'].reshape(1, DIM),
             p['conv1_b'].reshape(1, DIM), p['conv2_W'])


def _tc_post_body(sa, sb, g2, d, sW1, sb1_, sW2, sb2_, cb,
                  dW1, db1, dW2, db2, clW, clb, avW1, avb1, avW2, avb2,
                  cfW1, cfb1, cfW2, cfb2,
                  sp2_o, h2_o, xr_o, lg_o, ad_o, cf_o):
    deg = d[:, 0:1]
    dis = lax.rsqrt(deg + 1.0)
    agg = sa[:] / (deg + 1e-6)
    sp2 = _dot(_lk(_dot(agg, sW1[:]) + sb1_[:]), sW2[:]) + sb2_[:]
    h2 = _lk(dis * (sb[:] + g2[:]) + cb[:] + sp2)
    sp2_o[:] = sp2
    h2_o[:] = h2
    xr_o[:] = _dot(_lk(_dot(h2, dW1[:]) + db1[:]), dW2[:]) + db2[:]
    lg_o[:] = _dot(h2, clW[:]) + clb[:]
    ad_o[:] = _dot(_lk(_dot(h2, avW1[:]) + avb1[:]), avW2[:]) + avb2[:]
    cf_o[:] = _dot(_lk(_dot(h2, cfW1[:]) + cfb1[:]), cfW2[:]) + cfb2[:]


def _run_post(sa, sb, g2, d, p):
    f = pl.pallas_call(
        _tc_post_body,
        grid=(N // BLK,),
        in_specs=[
            _row(DIM), _row(DIM), _row(DIM), _row(DEGW),
            _full((DIM, 64)), _full((1, 64)), _full((64, DIM)), _full((1, DIM)),
            _full((1, DIM)),
            _full((DIM, DIM)), _full((1, DIM)), _full((DIM, DIM)), _full((1, DIM)),
            _full((DIM, 2)), _full((1, 2)),
            _full((DIM, 64)), _full((1, 64)), _full((64, 2)), _full((1, 2)),
            _full((DIM, DIM)), _full((1, DIM)), _full((DIM, DIM)), _full((1, DIM)),
        ],
        out_specs=[_row(DIM), _row(DIM), _row(DIM), _row(2), _row(2), _row(DIM)],
        out_shape=[
            jax.ShapeDtypeStruct((N, DIM), jnp.float32),
            jax.ShapeDtypeStruct((N, DIM), jnp.float32),
            jax.ShapeDtypeStruct((N, DIM), jnp.float32),
            jax.ShapeDtypeStruct((N, 2), jnp.float32),
            jax.ShapeDtypeStruct((N, 2), jnp.float32),
            jax.ShapeDtypeStruct((N, DIM), jnp.float32),
        ],
    )
    return f(sa, sb, g2, d,
             p['spg2_W1'], p['spg2_b1'].reshape(1, 64),
             p['spg2_W2'], p['spg2_b2'].reshape(1, DIM),
             p['conv2_b'].reshape(1, DIM),
             p['dec_W1'], p['dec_b1'].reshape(1, DIM),
             p['dec_W2'], p['dec_b2'].reshape(1, DIM),
             p['cls_W'], p['cls_b'].reshape(1, 2),
             p['adv_W1'], p['adv_b1'].reshape(1, 64),
             p['adv_W2'], p['adv_b2'].reshape(1, 2),
             p['cf_W1'], p['cf_b1'].reshape(1, DIM),
             p['cf_W2'], p['cf_b2'].reshape(1, DIM))


def kernel(x, edge_index, params):
    src, dst = edge_index[0], edge_index[1]
    d = _run_deg(src, dst)
    mask, xp, g1p = _run_pre(x, params)
    g1 = _run_scale(g1p, d)
    sa1, sb1 = _run_seg(xp, g1, src, dst)
    sp1, h1, g2 = _run_mid(sa1, sb1, g1, d, params)
    sa2, sb2 = _run_seg(h1, g2, src, dst)
    sp2, h2, x_recon, logits, adv_logits, cf = _run_post(sa2, sb2, g2, d,
                                                         params)
    return (x_recon, h2, mask, sp1, sp2, logits, adv_logits, cf)


# speedup vs baseline: 2.5380x; 1.7580x over previous
import jax
import jax.numpy as jnp
from jax import lax
from jax.experimental import pallas as pl
from jax.experimental.pallas import tpu as pltpu

N = 10000
DIM = 128
E = 320000
EB = 1024
EPAD = 327680
DEGW = 8
BLK = 1000


def _seg_body(sidx, didx, ta, tb, outa, outb):
    step = pl.program_id(0)

    @pl.when(step == 0)
    def _():
        outa[...] = jnp.zeros_like(outa)
        outb[...] = jnp.zeros_like(outb)

    def edge(i, c):
        s = sidx[i]
        d = didx[i]
        ra = ta[pl.ds(s, 1), :]
        rb = tb[pl.ds(s, 1), :]
        outa[pl.ds(d, 1), :] += ra
        outb[pl.ds(d, 1), :] += rb
        return c

    lax.fori_loop(0, EB, edge, 0, unroll=4)


def _deg_body(sidx, didx, deg):
    step = pl.program_id(0)

    @pl.when(step == 0)
    def _():
        deg[...] = jnp.zeros_like(deg)

    def edge(i, c):
        s = sidx[i]
        d = didx[i]
        one = jnp.full((1, DEGW), 1.0, jnp.float32) * jnp.where(
            s < N, 1.0, 0.0).astype(jnp.float32)
        deg[pl.ds(d, 1), :] += one
        return c

    lax.fori_loop(0, EB, edge, 0, unroll=4)


def _pad_edges(src, dst):
    src = jnp.concatenate([src, jnp.full((EPAD - E,), N, jnp.int32)])
    dst = jnp.concatenate([dst, jnp.zeros((EPAD - E,), jnp.int32)])
    return src, dst


def _run_deg(src, dst, interpret=False):
    src, dst = _pad_edges(src, dst)
    f = pl.pallas_call(
        _deg_body,
        grid=(EPAD // EB,),
        in_specs=[
            pl.BlockSpec((EB,), lambda i: (i,), memory_space=pltpu.SMEM),
            pl.BlockSpec((EB,), lambda i: (i,), memory_space=pltpu.SMEM),
        ],
        out_specs=[pl.BlockSpec((N, DEGW), lambda i: (0, 0))],
        out_shape=[jax.ShapeDtypeStruct((N, DEGW), jnp.float32)],
        interpret=interpret,
    )
    return f(src, dst)[0]


def _run_seg(ta, tb, src, dst, interpret=False):
    ta = jnp.concatenate([ta, jnp.zeros((1, DIM), jnp.float32)])
    tb = jnp.concatenate([tb, jnp.zeros((1, DIM), jnp.float32)])
    src, dst = _pad_edges(src, dst)
    f = pl.pallas_call(
        _seg_body,
        grid=(EPAD // EB,),
        in_specs=[
            pl.BlockSpec((EB,), lambda i: (i,), memory_space=pltpu.SMEM),
            pl.BlockSpec((EB,), lambda i: (i,), memory_space=pltpu.SMEM),
            pl.BlockSpec((N + 1, DIM), lambda i: (0, 0)),
            pl.BlockSpec((N + 1, DIM), lambda i: (0, 0)),
        ],
        out_specs=[
            pl.BlockSpec((N, DIM), lambda i: (0, 0)),
            pl.BlockSpec((N, DIM), lambda i: (0, 0)),
        ],
        out_shape=[
            jax.ShapeDtypeStruct((N, DIM), jnp.float32),
            jax.ShapeDtypeStruct((N, DIM), jnp.float32),
        ],
        interpret=interpret,
    )
    return f(src, dst, ta, tb)


def _lk(v):
    return jnp.where(v >= 0.0, v, 0.01 * v)


def _dot(a, b):
    return jnp.dot(a, b, preferred_element_type=jnp.float32)


def _row(w):
    return pl.BlockSpec((BLK, w), lambda i: (i, 0))


def _full(shape):
    return pl.BlockSpec(shape, lambda i: (0,) * len(shape))


def _tc_pre_body(x, aW1, ab1, aW2, ab2, cW, mask_o, xp_o, gp_o):
    h = _lk(_dot(x[:], aW1[:]) + ab1[:])
    m = jax.nn.sigmoid(_dot(h, aW2[:]) + ab2[:])
    xp = x[:] * m
    mask_o[:] = m
    xp_o[:] = xp
    gp_o[:] = _dot(xp, cW[:])


def _run_pre(x, p):
    f = pl.pallas_call(
        _tc_pre_body,
        grid=(N // BLK,),
        in_specs=[
            _row(DIM),
            _full((DIM, 64)), _full((1, 64)), _full((64, DIM)), _full((1, DIM)),
            _full((DIM, DIM)),
        ],
        out_specs=[_row(DIM)] * 3,
        out_shape=[jax.ShapeDtypeStruct((N, DIM), jnp.float32)] * 3,
    )
    return f(x, p['apg_W1'], p['apg_b1'].reshape(1, 64),
             p['apg_W2'], p['apg_b2'].reshape(1, DIM), p['conv1_W'])


def _tc_scale_body(gp, d, g_o):
    dis = lax.rsqrt(d[:, 0:1] + 1.0)
    g_o[:] = gp[:] * dis


def _run_scale(gp, d):
    f = pl.pallas_call(
        _tc_scale_body,
        grid=(N // BLK,),
        in_specs=[_row(DIM), _row(DEGW)],
        out_specs=[_row(DIM)],
        out_shape=[jax.ShapeDtypeStruct((N, DIM), jnp.float32)],
    )
    return f(gp, d)[0]


def _tc_mid_body(sa, sb, g1, d, sW1, sb1_, sW2, sb2_, cb, c2W,
                 sp1_o, h1_o, g2p_o):
    deg = d[:, 0:1]
    dis = lax.rsqrt(deg + 1.0)
    agg = sa[:] / (deg + 1e-6)
    sp1 = _dot(_lk(_dot(agg, sW1[:]) + sb1_[:]), sW2[:]) + sb2_[:]
    h1 = _lk(dis * (sb[:] + g1[:]) + cb[:] + sp1)
    sp1_o[:] = sp1
    h1_o[:] = h1
    g2p_o[:] = _dot(h1, c2W[:]) * dis


def _run_mid(sa, sb, g1, d, p):
    f = pl.pallas_call(
        _tc_mid_body,
        grid=(N // BLK,),
        in_specs=[
            _row(DIM), _row(DIM), _row(DIM), _row(DEGW),
            _full((DIM, 64)), _full((1, 64)), _full((64, DIM)), _full((1, DIM)),
            _full((1, DIM)), _full((DIM, DIM)),
        ],
        out_specs=[_row(DIM)] * 3,
        out_shape=[jax.ShapeDtypeStruct((N, DIM), jnp.float32)] * 3,
    )
    return f(sa, sb, g1, d,
             p['spg1_W1'], p['spg1_b1'].reshape(1, 64),
             p['spg1_W2'], p['spg1_b2'].reshape(1, DIM),
             p['conv1_b'].reshape(1, DIM), p['conv2_W'])


def _tc_post_body(sa, sb, g2, d, sW1, sb1_, sW2, sb2_, cb,
                  dW1, db1, dW2, db2, clW, clb, avW1, avb1, avW2, avb2,
                  cfW1, cfb1, cfW2, cfb2,
                  sp2_o, h2_o, xr_o, lg_o, ad_o, cf_o):
    deg = d[:, 0:1]
    dis = lax.rsqrt(deg + 1.0)
    agg = sa[:] / (deg + 1e-6)
    sp2 = _dot(_lk(_dot(agg, sW1[:]) + sb1_[:]), sW2[:]) + sb2_[:]
    h2 = _lk(dis * (sb[:] + g2[:]) + cb[:] + sp2)
    sp2_o[:] = sp2
    h2_o[:] = h2
    xr_o[:] = _dot(_lk(_dot(h2, dW1[:]) + db1[:]), dW2[:]) + db2[:]
    lg_o[:] = _dot(h2, clW[:]) + clb[:]
    ad_o[:] = _dot(_lk(_dot(h2, avW1[:]) + avb1[:]), avW2[:]) + avb2[:]
    cf_o[:] = _dot(_lk(_dot(h2, cfW1[:]) + cfb1[:]), cfW2[:]) + cfb2[:]


def _run_post(sa, sb, g2, d, p):
    f = pl.pallas_call(
        _tc_post_body,
        grid=(N // BLK,),
        in_specs=[
            _row(DIM), _row(DIM), _row(DIM), _row(DEGW),
            _full((DIM, 64)), _full((1, 64)), _full((64, DIM)), _full((1, DIM)),
            _full((1, DIM)),
            _full((DIM, DIM)), _full((1, DIM)), _full((DIM, DIM)), _full((1, DIM)),
            _full((DIM, 2)), _full((1, 2)),
            _full((DIM, 64)), _full((1, 64)), _full((64, 2)), _full((1, 2)),
            _full((DIM, DIM)), _full((1, DIM)), _full((DIM, DIM)), _full((1, DIM)),
        ],
        out_specs=[_row(DIM), _row(DIM), _row(DIM), _row(2), _row(2), _row(DIM)],
        out_shape=[
            jax.ShapeDtypeStruct((N, DIM), jnp.float32),
            jax.ShapeDtypeStruct((N, DIM), jnp.float32),
            jax.ShapeDtypeStruct((N, DIM), jnp.float32),
            jax.ShapeDtypeStruct((N, 2), jnp.float32),
            jax.ShapeDtypeStruct((N, 2), jnp.float32),
            jax.ShapeDtypeStruct((N, DIM), jnp.float32),
        ],
    )
    return f(sa, sb, g2, d,
             p['spg2_W1'], p['spg2_b1'].reshape(1, 64),
             p['spg2_W2'], p['spg2_b2'].reshape(1, DIM),
             p['conv2_b'].reshape(1, DIM),
             p['dec_W1'], p['dec_b1'].reshape(1, DIM),
             p['dec_W2'], p['dec_b2'].reshape(1, DIM),
             p['cls_W'], p['cls_b'].reshape(1, 2),
             p['adv_W1'], p['adv_b1'].reshape(1, 64),
             p['adv_W2'], p['adv_b2'].reshape(1, 2),
             p['cf_W1'], p['cf_b1'].reshape(1, DIM),
             p['cf_W2'], p['cf_b2'].reshape(1, DIM))


def kernel(x, edge_index, params):
    src, dst = edge_index[0], edge_index[1]
    d = _run_deg(src, dst)
    mask, xp, g1p = _run_pre(x, params)
    g1 = _run_scale(g1p, d)
    sa1, sb1 = _run_seg(xp, g1, src, dst)
    sp1, h1, g2 = _run_mid(sa1, sb1, g1, d, params)
    sa2, sb2 = _run_seg(h1, g2, src, dst)
    sp2, h2, x_recon, logits, adv_logits, cf = _run_post(sa2, sb2, g2, d,
                                                         params)
    return (x_recon, h2, mask, sp1, sp2, logits, adv_logits, cf)


# speedup vs baseline: 2.6696x; 1.0518x over previous
import jax
import jax.numpy as jnp
from jax import lax
from jax.experimental import pallas as pl
from jax.experimental.pallas import tpu as pltpu

N = 10000
DIM = 128
E = 320000
EB = 1024
EPAD = 327680
DEGW = 8
BLK = 1000


def _seg_body(sidx, didx, ta, tb, outa, outb):
    step = pl.program_id(0)

    @pl.when(step == 0)
    def _():
        outa[...] = jnp.zeros_like(outa)
        outb[...] = jnp.zeros_like(outb)

    def edge(i, c):
        s = sidx[i]
        d = didx[i]
        ra = ta[pl.ds(s, 1), :]
        rb = tb[pl.ds(s, 1), :]
        outa[pl.ds(d, 1), :] += ra
        outb[pl.ds(d, 1), :] += rb
        return c

    lax.fori_loop(0, EB, edge, 0, unroll=8)


def _deg_body(sidx, didx, deg):
    step = pl.program_id(0)

    @pl.when(step == 0)
    def _():
        deg[...] = jnp.zeros_like(deg)

    def edge(i, c):
        s = sidx[i]
        d = didx[i]
        one = jnp.full((1, DEGW), 1.0, jnp.float32) * jnp.where(
            s < N, 1.0, 0.0).astype(jnp.float32)
        deg[pl.ds(d, 1), :] += one
        return c

    lax.fori_loop(0, EB, edge, 0, unroll=8)


def _pad_edges(src, dst):
    src = jnp.concatenate([src, jnp.full((EPAD - E,), N, jnp.int32)])
    dst = jnp.concatenate([dst, jnp.zeros((EPAD - E,), jnp.int32)])
    return src, dst


def _run_deg(src, dst, interpret=False):
    src, dst = _pad_edges(src, dst)
    f = pl.pallas_call(
        _deg_body,
        grid=(EPAD // EB,),
        in_specs=[
            pl.BlockSpec((EB,), lambda i: (i,), memory_space=pltpu.SMEM),
            pl.BlockSpec((EB,), lambda i: (i,), memory_space=pltpu.SMEM),
        ],
        out_specs=[pl.BlockSpec((N, DEGW), lambda i: (0, 0))],
        out_shape=[jax.ShapeDtypeStruct((N, DEGW), jnp.float32)],
        interpret=interpret,
    )
    return f(src, dst)[0]


def _run_seg(ta, tb, src, dst, interpret=False):
    ta = jnp.concatenate([ta, jnp.zeros((1, DIM), jnp.float32)])
    tb = jnp.concatenate([tb, jnp.zeros((1, DIM), jnp.float32)])
    src, dst = _pad_edges(src, dst)
    f = pl.pallas_call(
        _seg_body,
        grid=(EPAD // EB,),
        in_specs=[
            pl.BlockSpec((EB,), lambda i: (i,), memory_space=pltpu.SMEM),
            pl.BlockSpec((EB,), lambda i: (i,), memory_space=pltpu.SMEM),
            pl.BlockSpec((N + 1, DIM), lambda i: (0, 0)),
            pl.BlockSpec((N + 1, DIM), lambda i: (0, 0)),
        ],
        out_specs=[
            pl.BlockSpec((N, DIM), lambda i: (0, 0)),
            pl.BlockSpec((N, DIM), lambda i: (0, 0)),
        ],
        out_shape=[
            jax.ShapeDtypeStruct((N, DIM), jnp.float32),
            jax.ShapeDtypeStruct((N, DIM), jnp.float32),
        ],
        interpret=interpret,
    )
    return f(src, dst, ta, tb)


def _lk(v):
    return jnp.where(v >= 0.0, v, 0.01 * v)


def _dot(a, b):
    return jnp.dot(a, b, preferred_element_type=jnp.float32)


def _row(w):
    return pl.BlockSpec((BLK, w), lambda i: (i, 0))


def _full(shape):
    return pl.BlockSpec(shape, lambda i: (0,) * len(shape))


def _tc_pre_body(x, aW1, ab1, aW2, ab2, cW, mask_o, xp_o, gp_o):
    h = _lk(_dot(x[:], aW1[:]) + ab1[:])
    m = jax.nn.sigmoid(_dot(h, aW2[:]) + ab2[:])
    xp = x[:] * m
    mask_o[:] = m
    xp_o[:] = xp
    gp_o[:] = _dot(xp, cW[:])


def _run_pre(x, p):
    f = pl.pallas_call(
        _tc_pre_body,
        grid=(N // BLK,),
        in_specs=[
            _row(DIM),
            _full((DIM, 64)), _full((1, 64)), _full((64, DIM)), _full((1, DIM)),
            _full((DIM, DIM)),
        ],
        out_specs=[_row(DIM)] * 3,
        out_shape=[jax.ShapeDtypeStruct((N, DIM), jnp.float32)] * 3,
    )
    return f(x, p['apg_W1'], p['apg_b1'].reshape(1, 64),
             p['apg_W2'], p['apg_b2'].reshape(1, DIM), p['conv1_W'])


def _tc_scale_body(gp, d, g_o):
    dis = lax.rsqrt(d[:, 0:1] + 1.0)
    g_o[:] = gp[:] * dis


def _run_scale(gp, d):
    f = pl.pallas_call(
        _tc_scale_body,
        grid=(N // BLK,),
        in_specs=[_row(DIM), _row(DEGW)],
        out_specs=[_row(DIM)],
        out_shape=[jax.ShapeDtypeStruct((N, DIM), jnp.float32)],
    )
    return f(gp, d)[0]


def _tc_mid_body(sa, sb, g1, d, sW1, sb1_, sW2, sb2_, cb, c2W,
                 sp1_o, h1_o, g2p_o):
    deg = d[:, 0:1]
    dis = lax.rsqrt(deg + 1.0)
    agg = sa[:] / (deg + 1e-6)
    sp1 = _dot(_lk(_dot(agg, sW1[:]) + sb1_[:]), sW2[:]) + sb2_[:]
    h1 = _lk(dis * (sb[:] + g1[:]) + cb[:] + sp1)
    sp1_o[:] = sp1
    h1_o[:] = h1
    g2p_o[:] = _dot(h1, c2W[:]) * dis


def _run_mid(sa, sb, g1, d, p):
    f = pl.pallas_call(
        _tc_mid_body,
        grid=(N // BLK,),
        in_specs=[
            _row(DIM), _row(DIM), _row(DIM), _row(DEGW),
            _full((DIM, 64)), _full((1, 64)), _full((64, DIM)), _full((1, DIM)),
            _full((1, DIM)), _full((DIM, DIM)),
        ],
        out_specs=[_row(DIM)] * 3,
        out_shape=[jax.ShapeDtypeStruct((N, DIM), jnp.float32)] * 3,
    )
    return f(sa, sb, g1, d,
             p['spg1_W1'], p['spg1_b1'].reshape(1, 64),
             p['spg1_W2'], p['spg1_b2'].reshape(1, DIM),
             p['conv1_b'].reshape(1, DIM), p['conv2_W'])


def _tc_post_body(sa, sb, g2, d, sW1, sb1_, sW2, sb2_, cb,
                  dW1, db1, dW2, db2, clW, clb, avW1, avb1, avW2, avb2,
                  cfW1, cfb1, cfW2, cfb2,
                  sp2_o, h2_o, xr_o, lg_o, ad_o, cf_o):
    deg = d[:, 0:1]
    dis = lax.rsqrt(deg + 1.0)
    agg = sa[:] / (deg + 1e-6)
    sp2 = _dot(_lk(_dot(agg, sW1[:]) + sb1_[:]), sW2[:]) + sb2_[:]
    h2 = _lk(dis * (sb[:] + g2[:]) + cb[:] + sp2)
    sp2_o[:] = sp2
    h2_o[:] = h2
    xr_o[:] = _dot(_lk(_dot(h2, dW1[:]) + db1[:]), dW2[:]) + db2[:]
    lg_o[:] = _dot(h2, clW[:]) + clb[:]
    ad_o[:] = _dot(_lk(_dot(h2, avW1[:]) + avb1[:]), avW2[:]) + avb2[:]
    cf_o[:] = _dot(_lk(_dot(h2, cfW1[:]) + cfb1[:]), cfW2[:]) + cfb2[:]


def _run_post(sa, sb, g2, d, p):
    f = pl.pallas_call(
        _tc_post_body,
        grid=(N // BLK,),
        in_specs=[
            _row(DIM), _row(DIM), _row(DIM), _row(DEGW),
            _full((DIM, 64)), _full((1, 64)), _full((64, DIM)), _full((1, DIM)),
            _full((1, DIM)),
            _full((DIM, DIM)), _full((1, DIM)), _full((DIM, DIM)), _full((1, DIM)),
            _full((DIM, 2)), _full((1, 2)),
            _full((DIM, 64)), _full((1, 64)), _full((64, 2)), _full((1, 2)),
            _full((DIM, DIM)), _full((1, DIM)), _full((DIM, DIM)), _full((1, DIM)),
        ],
        out_specs=[_row(DIM), _row(DIM), _row(DIM), _row(2), _row(2), _row(DIM)],
        out_shape=[
            jax.ShapeDtypeStruct((N, DIM), jnp.float32),
            jax.ShapeDtypeStruct((N, DIM), jnp.float32),
            jax.ShapeDtypeStruct((N, DIM), jnp.float32),
            jax.ShapeDtypeStruct((N, 2), jnp.float32),
            jax.ShapeDtypeStruct((N, 2), jnp.float32),
            jax.ShapeDtypeStruct((N, DIM), jnp.float32),
        ],
    )
    return f(sa, sb, g2, d,
             p['spg2_W1'], p['spg2_b1'].reshape(1, 64),
             p['spg2_W2'], p['spg2_b2'].reshape(1, DIM),
             p['conv2_b'].reshape(1, DIM),
             p['dec_W1'], p['dec_b1'].reshape(1, DIM),
             p['dec_W2'], p['dec_b2'].reshape(1, DIM),
             p['cls_W'], p['cls_b'].reshape(1, 2),
             p['adv_W1'], p['adv_b1'].reshape(1, 64),
             p['adv_W2'], p['adv_b2'].reshape(1, 2),
             p['cf_W1'], p['cf_b1'].reshape(1, DIM),
             p['cf_W2'], p['cf_b2'].reshape(1, DIM))


def kernel(x, edge_index, params):
    src, dst = edge_index[0], edge_index[1]
    d = _run_deg(src, dst)
    mask, xp, g1p = _run_pre(x, params)
    g1 = _run_scale(g1p, d)
    sa1, sb1 = _run_seg(xp, g1, src, dst)
    sp1, h1, g2 = _run_mid(sa1, sb1, g1, d, params)
    sa2, sb2 = _run_seg(h1, g2, src, dst)
    sp2, h2, x_recon, logits, adv_logits, cf = _run_post(sa2, sb2, g2, d,
                                                         params)
    return (x_recon, h2, mask, sp1, sp2, logits, adv_logits, cf)
